# Initial kernel scaffold; baseline (speedup 1.0000x reference)
import functools

import jax
import jax.numpy as jnp
from jax import lax
from jax.experimental import pallas as pl
from jax.experimental.pallas import tpu as pltpu
from jax.experimental.pallas import tpu_sc as plsc

N = 10000
E = 320000
D_IN = 128
HID = 64
H2 = 32

NC = 2
NS = 16
CH = 128
CHUNKS_PER_TILE = 80
E_PER_TILE = CHUNKS_PER_TILE * CH
E_PER_CORE = NS * E_PER_TILE
E_PAD = NC * E_PER_CORE
NACC = 10240
ROWS_PER_TILE = NACC // NS

_MESH = plsc.VectorSubcoreMesh(core_axis_name="c", subcore_axis_name="s")


@functools.partial(
    pl.kernel,
    mesh=_MESH,
    out_type=jax.ShapeDtypeStruct((NC, NACC), jnp.float32),
    scratch_types=[
        pltpu.VMEM((CH,), jnp.int32),
        pltpu.VMEM((CH,), jnp.float32),
        pltpu.VMEM_SHARED((NACC,), jnp.float32),
    ],
)
def _sc_degree(dst_hbm, ones_hbm, zeros_hbm, out_hbm, didx, ones_v, acc):
    c = lax.axis_index("c")
    s = lax.axis_index("s")
    pltpu.sync_copy(ones_hbm, ones_v)
    pltpu.sync_copy(zeros_hbm, acc.at[pl.ds(s * ROWS_PER_TILE, ROWS_PER_TILE)])
    plsc.subcore_barrier()
    base0 = c * E_PER_CORE + s * E_PER_TILE

    def body(i, carry):
        base = base0 + i * CH
        pltpu.sync_copy(dst_hbm.at[pl.ds(base, CH)], didx)
        pltpu.sync_copy(ones_v, acc.at[didx], add=True)
        return carry

    lax.fori_loop(0, CHUNKS_PER_TILE, body, 0)
    plsc.subcore_barrier()
    pltpu.sync_copy(
        acc.at[pl.ds(s * ROWS_PER_TILE, ROWS_PER_TILE)],
        out_hbm.at[c, pl.ds(s * ROWS_PER_TILE, ROWS_PER_TILE)],
    )


def _make_sc_agg(h):
    @functools.partial(
        pl.kernel,
        mesh=_MESH,
        out_type=jax.ShapeDtypeStruct((NC, NACC, h), jnp.float32),
        scratch_types=[
            pltpu.VMEM((CH,), jnp.int32),
            pltpu.VMEM((CH,), jnp.int32),
            pltpu.VMEM((CH, h), jnp.float32),
            pltpu.VMEM_SHARED((NACC, h), jnp.float32),
            pltpu.SemaphoreType.DMA,
        ],
    )
    def _sc_agg(src_hbm, dst_hbm, g_hbm, zeros_hbm, out_hbm,
                sidx, didx, rows, acc, sem):
        c = lax.axis_index("c")
        s = lax.axis_index("s")
        pltpu.sync_copy(zeros_hbm, acc.at[pl.ds(s * ROWS_PER_TILE, ROWS_PER_TILE)])
        plsc.subcore_barrier()
        base0 = c * E_PER_CORE + s * E_PER_TILE

        def body(i, carry):
            base = base0 + i * CH
            pltpu.sync_copy(src_hbm.at[pl.ds(base, CH)], sidx)
            pltpu.sync_copy(dst_hbm.at[pl.ds(base, CH)], didx)
            pltpu.async_copy(g_hbm.at[sidx], rows, sem).wait()
            pltpu.sync_copy(rows, acc.at[didx], add=True)
            return carry

        lax.fori_loop(0, CHUNKS_PER_TILE, body, 0)
        plsc.subcore_barrier()
        pltpu.sync_copy(
            acc.at[pl.ds(s * ROWS_PER_TILE, ROWS_PER_TILE)],
            out_hbm.at[c, pl.ds(s * ROWS_PER_TILE, ROWS_PER_TILE)],
        )

    return _sc_agg


_sc_agg64 = _make_sc_agg(HID)
_sc_agg32 = _make_sc_agg(H2)


def _tc_g1_body(x_ref, w_ref, dis_ref, o_ref):
    h = jnp.dot(x_ref[...], w_ref[...], preferred_element_type=jnp.float32)
    o_ref[...] = h * dis_ref[...]


def _tc_layer_body(ap_ref, g_ref, dis_ref, b_ref, w_ref, o_ref):
    ap = ap_ref[...]
    agg = ap[0, :N, :] + ap[1, :N, :] + g_ref[...]
    dis = dis_ref[...]
    hidden = jnp.maximum(agg * dis + b_ref[...], 0.0)
    o_ref[...] = jnp.dot(hidden, w_ref[...],
                         preferred_element_type=jnp.float32) * dis


def _tc_head_body(ap_ref, g_ref, dis_ref, b_ref, wh_ref, bh_ref, o_ref):
    ap = ap_ref[...]
    agg = ap[0, :N, :] + ap[1, :N, :] + g_ref[...]
    hidden = jnp.maximum(agg * dis_ref[...] + b_ref[...], 0.0)
    o_ref[...] = jnp.dot(hidden, wh_ref[...],
                         preferred_element_type=jnp.float32) + bh_ref[...]


def kernel(x, edge_index, W1, b1, W2, b2, Wh, bh):
    src = edge_index[0].astype(jnp.int32)
    dst = edge_index[1].astype(jnp.int32)
    pad = E_PAD - E
    src_p = jnp.concatenate([src, jnp.zeros((pad,), jnp.int32)])
    dst_p = jnp.concatenate([dst, jnp.full((pad,), N, jnp.int32)])

    ones_v = jnp.ones((CH,), jnp.float32)
    z1 = jnp.zeros((ROWS_PER_TILE,), jnp.float32)
    z64 = jnp.zeros((ROWS_PER_TILE, HID), jnp.float32)
    z32 = jnp.zeros((ROWS_PER_TILE, H2), jnp.float32)

    degp = _sc_degree(dst_p, ones_v, z1)
    deg = degp[0, :N] + degp[1, :N] + 1.0
    dis = lax.rsqrt(deg).reshape(N, 1)

    g1 = pl.pallas_call(
        _tc_g1_body,
        out_shape=jax.ShapeDtypeStruct((N, HID), jnp.float32),
    )(x, W1, dis)

    ap1 = _sc_agg64(src_p, dst_p, g1, z64)

    g2 = pl.pallas_call(
        _tc_layer_body,
        out_shape=jax.ShapeDtypeStruct((N, H2), jnp.float32),
    )(ap1, g1, dis, b1.reshape(1, HID), W2)

    ap2 = _sc_agg32(src_p, dst_p, g2, z32)

    out = pl.pallas_call(
        _tc_head_body,
        out_shape=jax.ShapeDtypeStruct((N, 1), jnp.float32),
    )(ap2, g2, dis, b2.reshape(1, H2), Wh, bh.reshape(1, 1))

    return out


# speedup vs baseline: 7.5205x; 7.5205x over previous
import functools

import jax
import jax.numpy as jnp
from jax import lax
from jax.experimental import pallas as pl
from jax.experimental.pallas import tpu as pltpu
from jax.experimental.pallas import tpu_sc as plsc

N = 10000
E = 320000
D_IN = 128
HID = 64
H2 = 32
W128 = 128

NC = 2
NS = 16
CH = 128
CHUNKS_PER_TILE = 80
E_PER_TILE = CHUNKS_PER_TILE * CH
E_PER_CORE = NS * E_PER_TILE
E_PAD = NC * E_PER_CORE
NACC = 10240
ROWS_PER_TILE = NACC // NS

_MESH = plsc.VectorSubcoreMesh(core_axis_name="c", subcore_axis_name="s")


@functools.partial(
    pl.kernel,
    mesh=_MESH,
    out_type=jax.ShapeDtypeStruct((NC, NACC), jnp.float32),
    scratch_types=[
        pltpu.VMEM((CH,), jnp.int32),
        pltpu.VMEM((CH,), jnp.float32),
        pltpu.VMEM_SHARED((NACC,), jnp.float32),
    ],
)
def _sc_degree(dst_hbm, zeros_hbm, out_hbm, didx, ones_v, acc):
    c = lax.axis_index("c")
    s = lax.axis_index("s")
    pltpu.sync_copy(
        zeros_hbm.at[pl.ds(0, ROWS_PER_TILE)],
        acc.at[pl.ds(s * ROWS_PER_TILE, ROWS_PER_TILE)],
    )
    for k in range(CH // 16):
        ones_v[pl.ds(k * 16, 16)] = jnp.ones((16,), jnp.float32)
    plsc.subcore_barrier()
    base0 = c * E_PER_CORE + s * E_PER_TILE

    def body(i, carry):
        base = base0 + i * CH
        pltpu.sync_copy(dst_hbm.at[pl.ds(base, CH)], didx)
        pltpu.sync_copy(ones_v, acc.at[didx], add=True)
        return carry

    lax.fori_loop(0, CHUNKS_PER_TILE, body, 0)
    plsc.subcore_barrier()
    pltpu.sync_copy(
        acc.at[pl.ds(s * ROWS_PER_TILE, ROWS_PER_TILE)],
        out_hbm.at[c, pl.ds(s * ROWS_PER_TILE, ROWS_PER_TILE)],
    )


@functools.partial(
    pl.kernel,
    mesh=_MESH,
    out_type=jax.ShapeDtypeStruct((NC, NACC, W128), jnp.float32),
    scratch_types=[
        pltpu.VMEM((CH,), jnp.int32),
        pltpu.VMEM((CH,), jnp.int32),
        pltpu.VMEM((CH, W128), jnp.float32),
        pltpu.VMEM_SHARED((NACC, W128), jnp.float32),
        pltpu.SemaphoreType.DMA,
    ],
)
def _sc_agg(src_hbm, dst_hbm, g_hbm, zeros_hbm, out_hbm,
            sidx, didx, rows, acc, sem):
    c = lax.axis_index("c")
    s = lax.axis_index("s")
    pltpu.sync_copy(zeros_hbm, acc.at[pl.ds(s * ROWS_PER_TILE, ROWS_PER_TILE)])
    plsc.subcore_barrier()
    base0 = c * E_PER_CORE + s * E_PER_TILE

    def body(i, carry):
        base = base0 + i * CH
        pltpu.sync_copy(src_hbm.at[pl.ds(base, CH)], sidx)
        pltpu.sync_copy(dst_hbm.at[pl.ds(base, CH)], didx)
        pltpu.async_copy(g_hbm.at[sidx], rows, sem).wait()
        pltpu.sync_copy(rows, acc.at[didx], add=True)
        return carry

    lax.fori_loop(0, CHUNKS_PER_TILE, body, 0)
    plsc.subcore_barrier()
    pltpu.sync_copy(
        acc.at[pl.ds(s * ROWS_PER_TILE, ROWS_PER_TILE)],
        out_hbm.at[c, pl.ds(s * ROWS_PER_TILE, ROWS_PER_TILE)],
    )


def _tc_g1_body(x_ref, w_ref, dis_ref, o_ref):
    h = jnp.dot(x_ref[...], w_ref[...], preferred_element_type=jnp.float32)
    o_ref[...] = h * dis_ref[...]


def _tc_layer_body(ap_ref, g_ref, dis_ref, b_ref, w_ref, o_ref):
    ap = ap_ref[...]
    agg = ap[0, :N, :] + ap[1, :N, :] + g_ref[...]
    dis = dis_ref[...]
    hidden = jnp.maximum(agg * dis + b_ref[...], 0.0)
    o_ref[...] = jnp.dot(hidden, w_ref[...],
                         preferred_element_type=jnp.float32) * dis


def _tc_head_body(ap_ref, g_ref, dis_ref, b_ref, wh_ref, bh_ref, o_ref):
    ap = ap_ref[...]
    agg = ap[0, :N, :] + ap[1, :N, :] + g_ref[...]
    hidden = jnp.maximum(agg * dis_ref[...] + b_ref[...], 0.0)
    o_ref[...] = jnp.dot(hidden, wh_ref[...],
                         preferred_element_type=jnp.float32) + bh_ref[...]


def kernel(x, edge_index, W1, b1, W2, b2, Wh, bh):
    src = edge_index[0].astype(jnp.int32)
    dst = edge_index[1].astype(jnp.int32)
    pad = E_PAD - E
    src_p = jnp.concatenate([src, jnp.zeros((pad,), jnp.int32)])
    dst_p = jnp.concatenate([dst, jnp.full((pad,), N, jnp.int32)])

    z1 = jnp.zeros((NACC,), jnp.float32)
    zrows = jnp.zeros((ROWS_PER_TILE, W128), jnp.float32)

    W1p = jnp.pad(W1, ((0, 0), (0, W128 - HID)))
    b1p = jnp.pad(b1, (0, W128 - HID)).reshape(1, W128)
    W2p = jnp.pad(W2, ((0, W128 - HID), (0, W128 - H2)))
    b2p = jnp.pad(b2, (0, W128 - H2)).reshape(1, W128)
    Whp = jnp.pad(Wh, ((0, W128 - H2), (0, 0)))

    degp = _sc_degree(dst_p, z1)
    deg = degp[0, :N] + degp[1, :N] + 1.0
    dis = lax.rsqrt(deg).reshape(N, 1)

    g1 = pl.pallas_call(
        _tc_g1_body,
        out_shape=jax.ShapeDtypeStruct((N, W128), jnp.float32),
    )(x, W1p, dis)

    ap1 = _sc_agg(src_p, dst_p, g1, zrows)

    g2 = pl.pallas_call(
        _tc_layer_body,
        out_shape=jax.ShapeDtypeStruct((N, W128), jnp.float32),
    )(ap1, g1, dis, b1p, W2p)

    ap2 = _sc_agg(src_p, dst_p, g2, zrows)

    out = pl.pallas_call(
        _tc_head_body,
        out_shape=jax.ShapeDtypeStruct((N, 1), jnp.float32),
    )(ap2, g2, dis, b2p, Whp, bh.reshape(1, 1))

    return out


# speedup vs baseline: 8.6139x; 1.1454x over previous
import functools

import jax
import jax.numpy as jnp
from jax import lax
from jax.experimental import pallas as pl
from jax.experimental.pallas import tpu as pltpu
from jax.experimental.pallas import tpu_sc as plsc

N = 10000
E = 320000
D_IN = 128
HID = 64
H2 = 32
W128 = 128

NC = 2
NS = 16
CH = 128
CHUNKS_PER_TILE = 80
E_PER_TILE = CHUNKS_PER_TILE * CH
E_PER_CORE = NS * E_PER_TILE
E_PAD = NC * E_PER_CORE
NACC = 10240
ROWS_PER_TILE = NACC // NS

_MESH = plsc.VectorSubcoreMesh(core_axis_name="c", subcore_axis_name="s")


@functools.partial(
    pl.kernel,
    mesh=_MESH,
    out_type=jax.ShapeDtypeStruct((NC, NACC), jnp.float32),
    scratch_types=[
        pltpu.VMEM((CHUNKS_PER_TILE, CH), jnp.int32),
        pltpu.VMEM((CH,), jnp.float32),
        pltpu.VMEM_SHARED((NACC,), jnp.float32),
        pltpu.SemaphoreType.DMA,
    ],
)
def _sc_degree(dst_hbm, zeros_hbm, out_hbm, didx, ones_v, acc, sem):
    c = lax.axis_index("c")
    s = lax.axis_index("s")
    row0 = (c * NS + s) * CHUNKS_PER_TILE
    pltpu.sync_copy(
        zeros_hbm.at[pl.ds(0, ROWS_PER_TILE)],
        acc.at[pl.ds(s * ROWS_PER_TILE, ROWS_PER_TILE)],
    )
    pltpu.sync_copy(dst_hbm.at[pl.ds(row0, CHUNKS_PER_TILE)], didx)
    for k in range(CH // 16):
        ones_v[pl.ds(k * 16, 16)] = jnp.ones((16,), jnp.float32)
    plsc.subcore_barrier()

    def body(i, carry):
        for k in range(8):
            pltpu.async_copy(ones_v, acc.at[didx.at[i * 8 + k]], sem, add=True)
        for k in range(8):
            pltpu.make_async_copy(ones_v, acc.at[didx.at[0]], sem).wait()
        return carry

    lax.fori_loop(0, CHUNKS_PER_TILE // 8, body, 0)
    plsc.subcore_barrier()
    pltpu.sync_copy(
        acc.at[pl.ds(s * ROWS_PER_TILE, ROWS_PER_TILE)],
        out_hbm.at[c, pl.ds(s * ROWS_PER_TILE, ROWS_PER_TILE)],
    )


@functools.partial(
    pl.kernel,
    mesh=_MESH,
    out_type=jax.ShapeDtypeStruct((NC, NACC, W128), jnp.float32),
    scratch_types=[
        pltpu.VMEM((CHUNKS_PER_TILE // 2, CH), jnp.int32),
        pltpu.VMEM((CHUNKS_PER_TILE // 2, CH), jnp.int32),
        pltpu.VMEM((2, CH, W128), jnp.float32),
        pltpu.VMEM_SHARED((NACC, W128), jnp.float32),
        pltpu.SemaphoreType.DMA,
        pltpu.SemaphoreType.DMA,
    ],
)
def _sc_agg(src_hbm, dst_hbm, g_hbm, zeros_hbm, out_hbm,
            sidx, didx, rows, acc, semg, sems):
    c = lax.axis_index("c")
    s = lax.axis_index("s")
    half = CHUNKS_PER_TILE // 2
    row0 = (c * NS + s) * CHUNKS_PER_TILE
    pltpu.sync_copy(zeros_hbm, acc.at[pl.ds(s * ROWS_PER_TILE, ROWS_PER_TILE)])
    plsc.subcore_barrier()

    for blk in range(2):
        pltpu.sync_copy(src_hbm.at[pl.ds(row0 + blk * half, half)], sidx)
        pltpu.sync_copy(dst_hbm.at[pl.ds(row0 + blk * half, half)], didx)
        pltpu.async_copy(g_hbm.at[sidx.at[0]], rows.at[0], semg).wait()

        def body(i, carry):
            p = lax.rem(i, 2)
            pn = lax.rem(i + 1, 2)
            gc = pltpu.async_copy(g_hbm.at[sidx.at[i + 1]], rows.at[pn], semg)
            sc = pltpu.async_copy(rows.at[p], acc.at[didx.at[i]], sems,
                                  add=True)
            sc.wait()
            gc.wait()
            return carry

        lax.fori_loop(0, half - 1, body, 0)
        pltpu.sync_copy(rows.at[(half - 1) % 2], acc.at[didx.at[half - 1]],
                        add=True)
    plsc.subcore_barrier()
    pltpu.sync_copy(
        acc.at[pl.ds(s * ROWS_PER_TILE, ROWS_PER_TILE)],
        out_hbm.at[c, pl.ds(s * ROWS_PER_TILE, ROWS_PER_TILE)],
    )


def _tc_g1_body(x_ref, w_ref, dis_ref, o_ref):
    h = jnp.dot(x_ref[...], w_ref[...], preferred_element_type=jnp.float32)
    o_ref[...] = h * dis_ref[...]


def _tc_layer_body(ap_ref, g_ref, dis_ref, b_ref, w_ref, o_ref):
    ap = ap_ref[...]
    agg = ap[0, :N, :] + ap[1, :N, :] + g_ref[...]
    dis = dis_ref[...]
    hidden = jnp.maximum(agg * dis + b_ref[...], 0.0)
    o_ref[...] = jnp.dot(hidden, w_ref[...],
                         preferred_element_type=jnp.float32) * dis


def _tc_head_body(ap_ref, g_ref, dis_ref, b_ref, wh_ref, bh_ref, o_ref):
    ap = ap_ref[...]
    agg = ap[0, :N, :] + ap[1, :N, :] + g_ref[...]
    hidden = jnp.maximum(agg * dis_ref[...] + b_ref[...], 0.0)
    o_ref[...] = jnp.dot(hidden, wh_ref[...],
                         preferred_element_type=jnp.float32) + bh_ref[...]


def kernel(x, edge_index, W1, b1, W2, b2, Wh, bh):
    src = edge_index[0].astype(jnp.int32)
    dst = edge_index[1].astype(jnp.int32)
    pad = E_PAD - E
    src_p = jnp.concatenate([src, jnp.zeros((pad,), jnp.int32)]).reshape(-1, CH)
    dst_p = jnp.concatenate([dst, jnp.full((pad,), N, jnp.int32)]).reshape(-1, CH)

    z1 = jnp.zeros((ROWS_PER_TILE,), jnp.float32)
    zrows = jnp.zeros((ROWS_PER_TILE, W128), jnp.float32)

    W1p = jnp.pad(W1, ((0, 0), (0, W128 - HID)))
    b1p = jnp.pad(b1, (0, W128 - HID)).reshape(1, W128)
    W2p = jnp.pad(W2, ((0, W128 - HID), (0, W128 - H2)))
    b2p = jnp.pad(b2, (0, W128 - H2)).reshape(1, W128)
    Whp = jnp.pad(Wh, ((0, W128 - H2), (0, 0)))

    degp = _sc_degree(dst_p, z1)
    deg = degp[0, :N] + degp[1, :N] + 1.0
    dis = lax.rsqrt(deg).reshape(N, 1)

    g1 = pl.pallas_call(
        _tc_g1_body,
        out_shape=jax.ShapeDtypeStruct((N, W128), jnp.float32),
    )(x, W1p, dis)

    ap1 = _sc_agg(src_p, dst_p, g1, zrows)

    g2 = pl.pallas_call(
        _tc_layer_body,
        out_shape=jax.ShapeDtypeStruct((N, W128), jnp.float32),
    )(ap1, g1, dis, b1p, W2p)

    ap2 = _sc_agg(src_p, dst_p, g2, zrows)

    out = pl.pallas_call(
        _tc_head_body,
        out_shape=jax.ShapeDtypeStruct((N, 1), jnp.float32),
    )(ap2, g2, dis, b2p, Whp, bh.reshape(1, 1))

    return out


# speedup vs baseline: 8.6502x; 1.0042x over previous
import functools

import jax
import jax.numpy as jnp
from jax import lax
from jax.experimental import pallas as pl
from jax.experimental.pallas import tpu as pltpu
from jax.experimental.pallas import tpu_sc as plsc

N = 10000
E = 320000
D_IN = 128
HID = 64
H2 = 32
W128 = 128

NC = 2
NS = 16
CH = 128
CHUNKS_PER_TILE = 80
E_PER_TILE = CHUNKS_PER_TILE * CH
E_PER_CORE = NS * E_PER_TILE
E_PAD = NC * E_PER_CORE
NACC = 10240
ROWS_PER_TILE = NACC // NS

_MESH = plsc.VectorSubcoreMesh(core_axis_name="c", subcore_axis_name="s")


@functools.partial(
    pl.kernel,
    mesh=_MESH,
    out_type=jax.ShapeDtypeStruct((NC, NACC), jnp.float32),
    scratch_types=[
        pltpu.VMEM((CHUNKS_PER_TILE, CH), jnp.int32),
        pltpu.VMEM((CH,), jnp.float32),
        pltpu.VMEM_SHARED((NACC,), jnp.float32),
        pltpu.SemaphoreType.DMA,
    ],
)
def _sc_degree(dst_hbm, zeros_hbm, out_hbm, didx, ones_v, acc, sem):
    c = lax.axis_index("c")
    s = lax.axis_index("s")
    row0 = (c * NS + s) * CHUNKS_PER_TILE
    pltpu.sync_copy(
        zeros_hbm.at[pl.ds(0, ROWS_PER_TILE)],
        acc.at[pl.ds(s * ROWS_PER_TILE, ROWS_PER_TILE)],
    )
    pltpu.sync_copy(dst_hbm.at[pl.ds(row0, CHUNKS_PER_TILE)], didx)
    for k in range(CH // 16):
        ones_v[pl.ds(k * 16, 16)] = jnp.ones((16,), jnp.float32)
    plsc.subcore_barrier()

    def body(i, carry):
        for k in range(8):
            pltpu.async_copy(ones_v, acc.at[didx.at[i * 8 + k]], sem, add=True)
        for k in range(8):
            pltpu.make_async_copy(ones_v, acc.at[didx.at[0]], sem).wait()
        return carry

    lax.fori_loop(0, CHUNKS_PER_TILE // 8, body, 0)
    plsc.subcore_barrier()
    pltpu.sync_copy(
        acc.at[pl.ds(s * ROWS_PER_TILE, ROWS_PER_TILE)],
        out_hbm.at[c, pl.ds(s * ROWS_PER_TILE, ROWS_PER_TILE)],
    )


@functools.partial(
    pl.kernel,
    mesh=_MESH,
    out_type=jax.ShapeDtypeStruct((NC, NACC, W128), jnp.float32),
    scratch_types=[
        pltpu.VMEM((CHUNKS_PER_TILE // 2, CH), jnp.int32),
        pltpu.VMEM((CHUNKS_PER_TILE // 2, CH), jnp.int32),
        pltpu.VMEM((2, CH, W128), jnp.float32),
        pltpu.VMEM_SHARED((NACC, W128), jnp.float32),
        pltpu.SemaphoreType.DMA,
        pltpu.SemaphoreType.DMA,
    ],
)
def _sc_agg(src_hbm, dst_hbm, g_hbm, zeros_hbm, out_hbm,
            sidx, didx, rows, acc, semg, sems):
    c = lax.axis_index("c")
    s = lax.axis_index("s")
    half = CHUNKS_PER_TILE // 2
    row0 = (c * NS + s) * CHUNKS_PER_TILE
    pltpu.sync_copy(zeros_hbm, acc.at[pl.ds(s * ROWS_PER_TILE, ROWS_PER_TILE)])
    plsc.subcore_barrier()

    for blk in range(2):
        pltpu.sync_copy(src_hbm.at[pl.ds(row0 + blk * half, half)], sidx)
        pltpu.sync_copy(dst_hbm.at[pl.ds(row0 + blk * half, half)], didx)
        pltpu.async_copy(g_hbm.at[sidx.at[0]], rows.at[0], semg).wait()

        def body(i, carry):
            p = lax.rem(i, 2)
            pn = lax.rem(i + 1, 2)
            gc = pltpu.async_copy(g_hbm.at[sidx.at[i + 1]], rows.at[pn], semg)
            sc = pltpu.async_copy(rows.at[p], acc.at[didx.at[i]], sems,
                                  add=True)
            sc.wait()
            gc.wait()
            return carry

        lax.fori_loop(0, half - 1, body, 0)
        pltpu.sync_copy(rows.at[(half - 1) % 2], acc.at[didx.at[half - 1]],
                        add=True)
    plsc.subcore_barrier()
    pltpu.sync_copy(
        acc.at[pl.ds(s * ROWS_PER_TILE, ROWS_PER_TILE)],
        out_hbm.at[c, pl.ds(s * ROWS_PER_TILE, ROWS_PER_TILE)],
    )


def _tc_g1_body(x_ref, w_ref, dis_ref, o_ref):
    h = jnp.dot(x_ref[...], w_ref[...], preferred_element_type=jnp.float32)
    o_ref[...] = h * dis_ref[...]


def _tc_layer_body(ap_ref, g_ref, dis_ref, b_ref, w_ref, o_ref):
    ap = ap_ref[...]
    agg = ap[0, :N, :] + ap[1, :N, :] + g_ref[...]
    dis = dis_ref[...]
    hidden = jnp.maximum(agg * dis + b_ref[...], 0.0)
    o_ref[...] = jnp.dot(hidden, w_ref[...],
                         preferred_element_type=jnp.float32) * dis


def _tc_head_body(ap_ref, g_ref, dis_ref, b_ref, wh_ref, bh_ref, o_ref):
    ap = ap_ref[...]
    agg = ap[0, :N, :] + ap[1, :N, :] + g_ref[...]
    hidden = jnp.maximum(agg * dis_ref[...] + b_ref[...], 0.0)
    o_ref[...] = jnp.dot(hidden, wh_ref[...],
                         preferred_element_type=jnp.float32) + bh_ref[...]


def kernel(x, edge_index, W1, b1, W2, b2, Wh, bh):
    src = edge_index[0].astype(jnp.int32)
    dst = edge_index[1].astype(jnp.int32)
    pad = E_PAD - E
    trash = N + jnp.arange(pad, dtype=jnp.int32) % (NACC - N)
    src_p = jnp.concatenate([src, jnp.zeros((pad,), jnp.int32)]).reshape(-1, CH)
    dst_p = jnp.concatenate([dst, trash]).reshape(-1, CH)

    z1 = jnp.zeros((ROWS_PER_TILE,), jnp.float32)
    zrows = jnp.zeros((ROWS_PER_TILE, W128), jnp.float32)

    W1p = jnp.pad(W1, ((0, 0), (0, W128 - HID)))
    b1p = jnp.pad(b1, (0, W128 - HID)).reshape(1, W128)
    W2p = jnp.pad(W2, ((0, W128 - HID), (0, W128 - H2)))
    b2p = jnp.pad(b2, (0, W128 - H2)).reshape(1, W128)
    Whp = jnp.pad(Wh, ((0, W128 - H2), (0, 0)))

    degp = _sc_degree(dst_p, z1)
    deg = degp[0, :N] + degp[1, :N] + 1.0
    dis = lax.rsqrt(deg).reshape(N, 1)

    g1 = pl.pallas_call(
        _tc_g1_body,
        out_shape=jax.ShapeDtypeStruct((N, W128), jnp.float32),
    )(x, W1p, dis)

    ap1 = _sc_agg(src_p, dst_p, g1, zrows)

    g2 = pl.pallas_call(
        _tc_layer_body,
        out_shape=jax.ShapeDtypeStruct((N, W128), jnp.float32),
    )(ap1, g1, dis, b1p, W2p)

    ap2 = _sc_agg(src_p, dst_p, g2, zrows)

    out = pl.pallas_call(
        _tc_head_body,
        out_shape=jax.ShapeDtypeStruct((N, 1), jnp.float32),
    )(ap2, g2, dis, b2p, Whp, bh.reshape(1, 1))

    return out
